# baseline (speedup 1.0000x reference)
import functools

import jax
import jax.numpy as jnp
from jax import lax
from jax.experimental import pallas as pl
from jax.experimental.pallas import tpu as pltpu

F32 = jnp.float32


def _cumsum_last(v):
    m, n = v.shape
    bs = n if n < 128 else 128
    nb = n // bs
    r = lax.broadcasted_iota(jnp.int32, (bs, bs), 0)
    c = lax.broadcasted_iota(jnp.int32, (bs, bs), 1)
    U = (r <= c).astype(F32)
    blocks = [
        jnp.dot(v[:, i * bs:(i + 1) * bs], U, preferred_element_type=F32)
        for i in range(nb)
    ]
    if nb == 1:
        return blocks[0]
    totals = jnp.concatenate([b[:, bs - 1:bs] for b in blocks], axis=1)
    r2 = lax.broadcasted_iota(jnp.int32, (nb, nb), 0)
    c2 = lax.broadcasted_iota(jnp.int32, (nb, nb), 1)
    U2 = (r2 <= c2).astype(F32)
    bcum = jnp.dot(totals, U2, preferred_element_type=F32)
    offs = bcum - totals
    return jnp.concatenate(
        [blocks[i] + offs[:, i:i + 1] for i in range(nb)], axis=1)


def _sqdist(a, b):
    ma = a.shape[0]
    an = jnp.sum(a * a, axis=-1, keepdims=True)
    bn = jnp.sum(b * b, axis=-1, keepdims=True)
    A2 = jnp.concatenate([a * (-2.0), jnp.ones((ma, 1), F32)], axis=1)
    B2 = jnp.concatenate([b, bn], axis=1)
    ab = lax.dot_general(A2, B2, (((1,), (1,)), ((), ())),
                         preferred_element_type=F32)
    return an + ab


def _first_index_where(cond, iota_row, n):
    return jnp.min(jnp.where(cond, iota_row, float(n)), axis=-1,
                   keepdims=True)


def _mlp(x, layers):
    for (W, b, g, bt) in layers:
        x = lax.dot_general(x, W, (((1,), (1,)), ((), ())),
                            preferred_element_type=F32)
        x = jnp.maximum(g * (x + b) + bt, 0.0)
    return x


def _fps_body(npoint, x0_ref, x1_ref, x2_ref, o0_ref, o1_ref, o2_ref):
    X0 = x0_ref[...]
    X1 = x1_ref[...]
    X2 = x2_ref[...]
    bsz, n = X0.shape
    iota_n = lax.broadcasted_iota(jnp.int32, (1, n), 1).astype(F32)
    iota_np = lax.broadcasted_iota(jnp.int32, (1, npoint), 1)

    def step(i, carry):
        distance, farthest, o0, o1, o2 = carry
        oh = (iota_n == farthest)
        c0 = jnp.sum(jnp.where(oh, X0, 0.0), axis=-1, keepdims=True)
        c1 = jnp.sum(jnp.where(oh, X1, 0.0), axis=-1, keepdims=True)
        c2 = jnp.sum(jnp.where(oh, X2, 0.0), axis=-1, keepdims=True)
        slot = iota_np == i
        o0 = jnp.where(slot, c0, o0)
        o1 = jnp.where(slot, c1, o1)
        o2 = jnp.where(slot, c2, o2)
        d = ((X0 - c0) * (X0 - c0) + (X1 - c1) * (X1 - c1)
             + (X2 - c2) * (X2 - c2))
        distance = jnp.minimum(distance, d)
        mx = jnp.max(distance, axis=-1, keepdims=True)
        farthest = _first_index_where(distance == mx, iota_n, n)
        return distance, farthest, o0, o1, o2

    init = (jnp.full((bsz, n), 1e10, F32), jnp.zeros((bsz, 1), F32),
            jnp.zeros((bsz, npoint), F32), jnp.zeros((bsz, npoint), F32),
            jnp.zeros((bsz, npoint), F32))
    _, _, o0, o1, o2 = lax.fori_loop(0, npoint, step, init)
    o0_ref[...] = o0
    o1_ref[...] = o1
    o2_ref[...] = o2


def _fps(xyz, npoint):
    bsz, n, _ = xyz.shape
    outs = pl.pallas_call(
        functools.partial(_fps_body, npoint),
        out_shape=[jax.ShapeDtypeStruct((bsz, npoint), F32)] * 3,
    )(xyz[:, :, 0], xyz[:, :, 1], xyz[:, :, 2])
    return jnp.stack(outs, axis=-1)


def _sa_body(r2, nsample, nlayers, xyz_ref, nx_ref, pts_ref, *wrefs, out_ref):
    layers = [tuple(wrefs[4 * i + j][...] for j in range(4))
              for i in range(nlayers)]
    X = xyz_ref[0]
    NX = nx_ref[0]
    P = pts_ref[0]
    feats = jnp.concatenate([X, P], axis=-1)

    n = X.shape[0]
    d = _sqdist(NX, X)
    valid = d <= r2
    total = jnp.sum(valid.astype(F32), axis=-1, keepdims=True)
    iota_n = lax.broadcasted_iota(jnp.int32, (1, n), 1).astype(F32)
    avail0 = jnp.where(valid, iota_n, float(n))
    cout = out_ref.shape[-1]
    sc = NX.shape[0]

    def body(k, carry):
        maxed, avail = carry
        jmin = jnp.min(avail, axis=-1, keepdims=True)
        oh = avail == jmin
        gk = jnp.dot(jnp.where(oh, 1.0, 0.0), feats,
                     preferred_element_type=F32)
        avail = jnp.where(oh, float(n), avail)
        x = jnp.concatenate([gk[:, :3] - NX, gk[:, 3:]], axis=-1)
        x = _mlp(x, layers)
        keep = total >= (k + 1).astype(F32)
        return jnp.where(keep, jnp.maximum(maxed, x), maxed), avail

    kmax = jnp.minimum(jnp.max(total).astype(jnp.int32), nsample)
    out_ref[0] = lax.fori_loop(
        0, kmax, body, (jnp.zeros((sc, cout), F32), avail0))[0]


def _sa(xyz, new_xyz, pts, layers, radius, nsample, s_chunk):
    bsz, n, _ = xyz.shape
    s = new_xyz.shape[1]
    ch = pts.shape[-1]
    cout = layers[-1][0].shape[0]
    wargs = []
    wspecs = []
    for (W, b, g, bt) in layers:
        o, i = W.shape
        wargs += [W, b.reshape(1, o), g.reshape(1, o), bt.reshape(1, o)]
        wspecs += [
            pl.BlockSpec((o, i), lambda bb, ss: (0, 0)),
            pl.BlockSpec((1, o), lambda bb, ss: (0, 0)),
            pl.BlockSpec((1, o), lambda bb, ss: (0, 0)),
            pl.BlockSpec((1, o), lambda bb, ss: (0, 0)),
        ]
    grid = (bsz, s // s_chunk)
    body = functools.partial(_sa_body, radius * radius, nsample, len(layers))

    def kern(*refs):
        body(*refs[:-1], out_ref=refs[-1])

    return pl.pallas_call(
        kern,
        grid=grid,
        in_specs=[
            pl.BlockSpec((1, n, 3), lambda bb, ss: (bb, 0, 0)),
            pl.BlockSpec((1, s_chunk, 3), lambda bb, ss: (bb, ss, 0)),
            pl.BlockSpec((1, n, ch), lambda bb, ss: (bb, 0, 0)),
        ] + wspecs,
        out_specs=pl.BlockSpec((1, s_chunk, cout), lambda bb, ss: (bb, ss, 0)),
        out_shape=jax.ShapeDtypeStruct((bsz, s, cout), F32),
        compiler_params=pltpu.CompilerParams(
            dimension_semantics=("parallel", "parallel")),
    )(xyz, new_xyz, pts, *wargs)


def _fp_body(nlayers, nhead, has_p1, x1_ref, x2_ref, p2_ref, *rest,
             out_ref):
    idx = 0
    if has_p1:
        p1_ref = rest[0]
        idx = 1
    wrefs = rest[idx:]
    layers = [tuple(wrefs[4 * i + j][...] for j in range(4))
              for i in range(nlayers + nhead)]
    cw = wrefs[4 * (nlayers + nhead)][...]
    cb = wrefs[4 * (nlayers + nhead) + 1][...]

    X1 = x1_ref[0]
    X2 = x2_ref[0]
    P2 = p2_ref[0]
    n1c = X1.shape[0]
    n2 = X2.shape[0]

    d = _sqdist(X1, X2)
    iota2 = lax.broadcasted_iota(jnp.int32, (1, n2), 1).astype(F32)
    W = jnp.zeros((n1c, n2), F32)
    wsum = jnp.zeros((n1c, 1), F32)
    dw = d
    for _ in range(3):
        m = jnp.min(dw, axis=-1, keepdims=True)
        am = _first_index_where(dw == m, iota2, n2)
        oh = (iota2 == am)
        rec = 1.0 / (m + 1e-8)
        W = W + jnp.where(oh, rec, 0.0)
        wsum = wsum + rec
        dw = jnp.where(oh, 1e30, dw)
    interp = jnp.dot(W, P2, preferred_element_type=F32) / wsum

    if has_p1:
        x = jnp.concatenate([p1_ref[0], interp], axis=-1)
    else:
        x = interp
    x = _mlp(x, layers)
    if nhead:
        x = lax.dot_general(cw, x, (((1,), (1,)), ((), ())),
                            preferred_element_type=F32) + cb
    out_ref[0] = x


def _fp(x1, x2, p1, p2, layers, n1_chunk, head_layers=None, cls=None):
    bsz, n1, _ = x1.shape
    n2 = x2.shape[1]
    c2 = p2.shape[-1]
    head_layers = head_layers or []
    all_layers = list(layers) + list(head_layers)
    if cls is not None:
        Wc, bc = cls
        cout = Wc.shape[0]
        cls_args = [Wc, bc.reshape(cout, 1)]
    else:
        cout = all_layers[-1][0].shape[0]
        cls_args = [jnp.zeros((1, 1), F32), jnp.zeros((1, 1), F32)]

    wargs = []
    wspecs = []
    for (W, b, g, bt) in all_layers:
        o, i = W.shape
        wargs += [W, b.reshape(1, o), g.reshape(1, o), bt.reshape(1, o)]
        wspecs += [
            pl.BlockSpec((o, i), lambda bb, ss: (0, 0)),
            pl.BlockSpec((1, o), lambda bb, ss: (0, 0)),
            pl.BlockSpec((1, o), lambda bb, ss: (0, 0)),
            pl.BlockSpec((1, o), lambda bb, ss: (0, 0)),
        ]
    wargs += cls_args
    wspecs += [pl.BlockSpec(cls_args[0].shape, lambda bb, ss: (0, 0)),
               pl.BlockSpec(cls_args[1].shape, lambda bb, ss: (0, 0))]

    in_arrays = [x1, x2, p2]
    in_specs = [
        pl.BlockSpec((1, n1_chunk, 3), lambda bb, ss: (bb, ss, 0)),
        pl.BlockSpec((1, n2, 3), lambda bb, ss: (bb, 0, 0)),
        pl.BlockSpec((1, n2, c2), lambda bb, ss: (bb, 0, 0)),
    ]
    has_p1 = p1 is not None
    if has_p1:
        c1 = p1.shape[-1]
        in_arrays.append(p1)
        in_specs.append(pl.BlockSpec((1, n1_chunk, c1),
                                     lambda bb, ss: (bb, ss, 0)))
    body = functools.partial(_fp_body, len(layers), len(head_layers), has_p1)

    def kern(*refs):
        body(*refs[:-1], out_ref=refs[-1])

    if cls is not None:
        out_spec = pl.BlockSpec((1, cout, n1_chunk), lambda bb, ss: (bb, 0, ss))
        out_shape = jax.ShapeDtypeStruct((bsz, cout, n1), F32)
    else:
        out_spec = pl.BlockSpec((1, n1_chunk, cout), lambda bb, ss: (bb, ss, 0))
        out_shape = jax.ShapeDtypeStruct((bsz, n1, cout), F32)
    return pl.pallas_call(
        kern,
        grid=(bsz, n1 // n1_chunk),
        in_specs=in_specs + wspecs,
        out_specs=out_spec,
        out_shape=out_shape,
        compiler_params=pltpu.CompilerParams(
            dimension_semantics=("parallel", "parallel")),
    )(*in_arrays, *wargs)


def kernel(xyz, params):
    x0 = jnp.transpose(xyz, (0, 2, 1))
    l0x = x0[..., :3]
    p0 = x0

    nx1 = _fps(l0x, 1024)
    l1p = _sa(l0x, nx1, p0, params['sa1'], 0.1, 32, s_chunk=256)
    nx2 = _fps(nx1, 256)
    l2p = _sa(nx1, nx2, l1p, params['sa2'], 0.2, 32, s_chunk=256)
    nx3 = _fps(nx2, 64)
    l3p = _sa(nx2, nx3, l2p, params['sa3'], 0.4, 32, s_chunk=64)
    nx4 = _fps(nx3, 16)
    l4p = _sa(nx3, nx4, l3p, params['sa4'], 0.8, 32, s_chunk=16)

    l3p = _fp(nx3, nx4, l3p, l4p, params['fp4'], n1_chunk=64)
    l2p = _fp(nx2, nx3, l2p, l3p, params['fp3'], n1_chunk=256)
    l1p = _fp(nx1, nx2, l1p, l2p, params['fp2'], n1_chunk=1024)
    return _fp(l0x, nx1, None, l1p, params['fp1'], n1_chunk=1024,
               head_layers=params['head'], cls=params['out'])


# speedup vs baseline: 1.1252x; 1.1252x over previous
import functools

import jax
import jax.numpy as jnp
from jax import lax
from jax.experimental import pallas as pl
from jax.experimental.pallas import tpu as pltpu

F32 = jnp.float32


def _cumsum_last(v):
    m, n = v.shape
    bs = n if n < 128 else 128
    nb = n // bs
    r = lax.broadcasted_iota(jnp.int32, (bs, bs), 0)
    c = lax.broadcasted_iota(jnp.int32, (bs, bs), 1)
    U = (r <= c).astype(F32)
    blocks = [
        jnp.dot(v[:, i * bs:(i + 1) * bs], U, preferred_element_type=F32)
        for i in range(nb)
    ]
    if nb == 1:
        return blocks[0]
    totals = jnp.concatenate([b[:, bs - 1:bs] for b in blocks], axis=1)
    r2 = lax.broadcasted_iota(jnp.int32, (nb, nb), 0)
    c2 = lax.broadcasted_iota(jnp.int32, (nb, nb), 1)
    U2 = (r2 <= c2).astype(F32)
    bcum = jnp.dot(totals, U2, preferred_element_type=F32)
    offs = bcum - totals
    return jnp.concatenate(
        [blocks[i] + offs[:, i:i + 1] for i in range(nb)], axis=1)


def _sqdist(a, b):
    ma = a.shape[0]
    an = jnp.sum(a * a, axis=-1, keepdims=True)
    bn = jnp.sum(b * b, axis=-1, keepdims=True)
    A2 = jnp.concatenate([a * (-2.0), jnp.ones((ma, 1), F32)], axis=1)
    B2 = jnp.concatenate([b, bn], axis=1)
    ab = lax.dot_general(A2, B2, (((1,), (1,)), ((), ())),
                         preferred_element_type=F32)
    return an + ab


def _first_index_where(cond, iota_row, n):
    return jnp.min(jnp.where(cond, iota_row, float(n)), axis=-1,
                   keepdims=True)


def _mlp(x, layers):
    for (W, b, g, bt) in layers:
        x = lax.dot_general(x, W, (((1,), (1,)), ((), ())),
                            preferred_element_type=F32)
        x = jnp.maximum(g * (x + b) + bt, 0.0)
    return x


def _fps_body(npoint, x0_ref, x1_ref, x2_ref, o0_ref, o1_ref, o2_ref):
    X0 = x0_ref[...]
    X1 = x1_ref[...]
    X2 = x2_ref[...]
    bsz, n = X0.shape
    iota_n = lax.broadcasted_iota(jnp.int32, (1, n), 1).astype(F32)
    iota_np = lax.broadcasted_iota(jnp.int32, (1, npoint), 1)

    def step(i, carry):
        distance, farthest, o0, o1, o2 = carry
        oh = (iota_n == farthest)
        c0 = jnp.sum(jnp.where(oh, X0, 0.0), axis=-1, keepdims=True)
        c1 = jnp.sum(jnp.where(oh, X1, 0.0), axis=-1, keepdims=True)
        c2 = jnp.sum(jnp.where(oh, X2, 0.0), axis=-1, keepdims=True)
        slot = iota_np == i
        o0 = jnp.where(slot, c0, o0)
        o1 = jnp.where(slot, c1, o1)
        o2 = jnp.where(slot, c2, o2)
        d = ((X0 - c0) * (X0 - c0) + (X1 - c1) * (X1 - c1)
             + (X2 - c2) * (X2 - c2))
        distance = jnp.minimum(distance, d)
        mx = jnp.max(distance, axis=-1, keepdims=True)
        farthest = _first_index_where(distance == mx, iota_n, n)
        return distance, farthest, o0, o1, o2

    init = (jnp.full((bsz, n), 1e10, F32), jnp.zeros((bsz, 1), F32),
            jnp.zeros((bsz, npoint), F32), jnp.zeros((bsz, npoint), F32),
            jnp.zeros((bsz, npoint), F32))
    _, _, o0, o1, o2 = lax.fori_loop(0, npoint, step, init)
    o0_ref[...] = o0
    o1_ref[...] = o1
    o2_ref[...] = o2


def _fps(xyz, npoint):
    bsz, n, _ = xyz.shape
    outs = pl.pallas_call(
        functools.partial(_fps_body, npoint),
        out_shape=[jax.ShapeDtypeStruct((bsz, npoint), F32)] * 3,
    )(xyz[:, :, 0], xyz[:, :, 1], xyz[:, :, 2])
    return jnp.stack(outs, axis=-1)


def _sa_body(r2, nsample, nlayers, xyz_ref, nx_ref, pts_ref, *wrefs, out_ref):
    layers = [tuple(wrefs[4 * i + j][...] for j in range(4))
              for i in range(nlayers)]
    X = xyz_ref[0]
    NX = nx_ref[0]
    P = pts_ref[0]
    feats = jnp.concatenate([X, P], axis=-1)

    d = _sqdist(NX, X)
    valid = d <= r2
    cnt = _cumsum_last(valid.astype(F32))
    total = cnt[:, -1:]
    key = jnp.where(valid, cnt, 0.0)
    cout = out_ref.shape[-1]
    sc = NX.shape[0]

    def body(k, maxed):
        kf = (k + 1).astype(F32)
        oh = jnp.where(key == kf, 1.0, 0.0)
        gk = jnp.dot(oh, feats, preferred_element_type=F32)
        x = jnp.concatenate([gk[:, :3] - NX, gk[:, 3:]], axis=-1)
        x = _mlp(x, layers)
        has = total >= kf
        return jnp.where(has, jnp.maximum(maxed, x), maxed)

    kmax = jnp.minimum(jnp.max(total).astype(jnp.int32), nsample)
    out_ref[0] = lax.fori_loop(0, kmax, body, jnp.zeros((sc, cout), F32))


def _sa(xyz, new_xyz, pts, layers, radius, nsample, s_chunk):
    bsz, n, _ = xyz.shape
    s = new_xyz.shape[1]
    ch = pts.shape[-1]
    cout = layers[-1][0].shape[0]
    wargs = []
    wspecs = []
    for (W, b, g, bt) in layers:
        o, i = W.shape
        wargs += [W, b.reshape(1, o), g.reshape(1, o), bt.reshape(1, o)]
        wspecs += [
            pl.BlockSpec((o, i), lambda bb, ss: (0, 0)),
            pl.BlockSpec((1, o), lambda bb, ss: (0, 0)),
            pl.BlockSpec((1, o), lambda bb, ss: (0, 0)),
            pl.BlockSpec((1, o), lambda bb, ss: (0, 0)),
        ]
    grid = (bsz, s // s_chunk)
    body = functools.partial(_sa_body, radius * radius, nsample, len(layers))

    def kern(*refs):
        body(*refs[:-1], out_ref=refs[-1])

    return pl.pallas_call(
        kern,
        grid=grid,
        in_specs=[
            pl.BlockSpec((1, n, 3), lambda bb, ss: (bb, 0, 0)),
            pl.BlockSpec((1, s_chunk, 3), lambda bb, ss: (bb, ss, 0)),
            pl.BlockSpec((1, n, ch), lambda bb, ss: (bb, 0, 0)),
        ] + wspecs,
        out_specs=pl.BlockSpec((1, s_chunk, cout), lambda bb, ss: (bb, ss, 0)),
        out_shape=jax.ShapeDtypeStruct((bsz, s, cout), F32),
        compiler_params=pltpu.CompilerParams(
            dimension_semantics=("parallel", "parallel")),
    )(xyz, new_xyz, pts, *wargs)


def _fp_body(nlayers, nhead, has_p1, x1_ref, x2_ref, p2_ref, *rest,
             out_ref):
    idx = 0
    if has_p1:
        p1_ref = rest[0]
        idx = 1
    wrefs = rest[idx:]
    layers = [tuple(wrefs[4 * i + j][...] for j in range(4))
              for i in range(nlayers + nhead)]
    cw = wrefs[4 * (nlayers + nhead)][...]
    cb = wrefs[4 * (nlayers + nhead) + 1][...]

    X1 = x1_ref[0]
    X2 = x2_ref[0]
    P2 = p2_ref[0]
    n1c = X1.shape[0]
    n2 = X2.shape[0]

    d = _sqdist(X1, X2)
    iota2 = lax.broadcasted_iota(jnp.int32, (1, n2), 1).astype(F32)
    W = jnp.zeros((n1c, n2), F32)
    wsum = jnp.zeros((n1c, 1), F32)
    dw = d
    for _ in range(3):
        m = jnp.min(dw, axis=-1, keepdims=True)
        am = _first_index_where(dw == m, iota2, n2)
        oh = (iota2 == am)
        rec = 1.0 / (m + 1e-8)
        W = W + jnp.where(oh, rec, 0.0)
        wsum = wsum + rec
        dw = jnp.where(oh, 1e30, dw)
    interp = jnp.dot(W, P2, preferred_element_type=F32) / wsum

    if has_p1:
        x = jnp.concatenate([p1_ref[0], interp], axis=-1)
    else:
        x = interp
    x = _mlp(x, layers)
    if nhead:
        x = lax.dot_general(cw, x, (((1,), (1,)), ((), ())),
                            preferred_element_type=F32) + cb
    out_ref[0] = x


def _fp(x1, x2, p1, p2, layers, n1_chunk, head_layers=None, cls=None):
    bsz, n1, _ = x1.shape
    n2 = x2.shape[1]
    c2 = p2.shape[-1]
    head_layers = head_layers or []
    all_layers = list(layers) + list(head_layers)
    if cls is not None:
        Wc, bc = cls
        cout = Wc.shape[0]
        cls_args = [Wc, bc.reshape(cout, 1)]
    else:
        cout = all_layers[-1][0].shape[0]
        cls_args = [jnp.zeros((1, 1), F32), jnp.zeros((1, 1), F32)]

    wargs = []
    wspecs = []
    for (W, b, g, bt) in all_layers:
        o, i = W.shape
        wargs += [W, b.reshape(1, o), g.reshape(1, o), bt.reshape(1, o)]
        wspecs += [
            pl.BlockSpec((o, i), lambda bb, ss: (0, 0)),
            pl.BlockSpec((1, o), lambda bb, ss: (0, 0)),
            pl.BlockSpec((1, o), lambda bb, ss: (0, 0)),
            pl.BlockSpec((1, o), lambda bb, ss: (0, 0)),
        ]
    wargs += cls_args
    wspecs += [pl.BlockSpec(cls_args[0].shape, lambda bb, ss: (0, 0)),
               pl.BlockSpec(cls_args[1].shape, lambda bb, ss: (0, 0))]

    in_arrays = [x1, x2, p2]
    in_specs = [
        pl.BlockSpec((1, n1_chunk, 3), lambda bb, ss: (bb, ss, 0)),
        pl.BlockSpec((1, n2, 3), lambda bb, ss: (bb, 0, 0)),
        pl.BlockSpec((1, n2, c2), lambda bb, ss: (bb, 0, 0)),
    ]
    has_p1 = p1 is not None
    if has_p1:
        c1 = p1.shape[-1]
        in_arrays.append(p1)
        in_specs.append(pl.BlockSpec((1, n1_chunk, c1),
                                     lambda bb, ss: (bb, ss, 0)))
    body = functools.partial(_fp_body, len(layers), len(head_layers), has_p1)

    def kern(*refs):
        body(*refs[:-1], out_ref=refs[-1])

    if cls is not None:
        out_spec = pl.BlockSpec((1, cout, n1_chunk), lambda bb, ss: (bb, 0, ss))
        out_shape = jax.ShapeDtypeStruct((bsz, cout, n1), F32)
    else:
        out_spec = pl.BlockSpec((1, n1_chunk, cout), lambda bb, ss: (bb, ss, 0))
        out_shape = jax.ShapeDtypeStruct((bsz, n1, cout), F32)
    return pl.pallas_call(
        kern,
        grid=(bsz, n1 // n1_chunk),
        in_specs=in_specs + wspecs,
        out_specs=out_spec,
        out_shape=out_shape,
        compiler_params=pltpu.CompilerParams(
            dimension_semantics=("parallel", "parallel")),
    )(*in_arrays, *wargs)


def kernel(xyz, params):
    x0 = jnp.transpose(xyz, (0, 2, 1))
    l0x = x0[..., :3]
    p0 = x0

    nx1 = _fps(l0x, 1024)
    l1p = _sa(l0x, nx1, p0, params['sa1'], 0.1, 32, s_chunk=512)
    nx2 = _fps(nx1, 256)
    l2p = _sa(nx1, nx2, l1p, params['sa2'], 0.2, 32, s_chunk=256)
    nx3 = _fps(nx2, 64)
    l3p = _sa(nx2, nx3, l2p, params['sa3'], 0.4, 32, s_chunk=64)
    nx4 = _fps(nx3, 16)
    l4p = _sa(nx3, nx4, l3p, params['sa4'], 0.8, 32, s_chunk=16)

    l3p = _fp(nx3, nx4, l3p, l4p, params['fp4'], n1_chunk=64)
    l2p = _fp(nx2, nx3, l2p, l3p, params['fp3'], n1_chunk=256)
    l1p = _fp(nx1, nx2, l1p, l2p, params['fp2'], n1_chunk=1024)
    return _fp(l0x, nx1, None, l1p, params['fp1'], n1_chunk=2048,
               head_layers=params['head'], cls=params['out'])
